# baseline (speedup 1.0000x reference)
import functools

import jax
import jax.numpy as jnp
from jax.experimental import pallas as pl
from jax.experimental.pallas import tpu as pltpu

HIDDEN = 2048
PROJ = 768
VOCAB = 100000
ROWS = 256
BV = 4096


def _mlp_head_kernel(t_ref, w1_ref, b1_ref, w2_ref, b2_ref, out_ref, x_ref):
    i = pl.program_id(0)

    @pl.when(i == 0)
    def _():
        p = jax.lax.dot_general(
            t_ref[...], w1_ref[...], (((1,), (0,)), ((), ())),
            precision=jax.lax.Precision.HIGHEST,
            preferred_element_type=jnp.float32,
        ) + b1_ref[...]
        x_ref[...] = 0.5 * p * (1.0 + jax.lax.erf(p * 0.7071067811865476))

    acc = jax.lax.dot_general(
        x_ref[...].astype(jnp.bfloat16),
        w2_ref[...].astype(jnp.bfloat16),
        (((1,), (0,)), ((), ())),
        preferred_element_type=jnp.float32,
    )
    out_ref[...] = acc + b2_ref[...]


@functools.partial(jax.jit, static_argnames=())
def kernel(t, W1, b1, W2, b2):
    B, S, _ = t.shape
    t2 = t.reshape(B * S, HIDDEN)
    nv = pl.cdiv(VOCAB, BV)
    out = pl.pallas_call(
        _mlp_head_kernel,
        grid=(nv,),
        in_specs=[
            pl.BlockSpec((ROWS, HIDDEN), lambda i: (0, 0)),
            pl.BlockSpec((HIDDEN, PROJ), lambda i: (0, 0)),
            pl.BlockSpec((1, PROJ), lambda i: (0, 0)),
            pl.BlockSpec((PROJ, BV), lambda i: (0, i)),
            pl.BlockSpec((1, BV), lambda i: (0, i)),
        ],
        out_specs=pl.BlockSpec((ROWS, BV), lambda i: (0, i)),
        out_shape=jax.ShapeDtypeStruct((ROWS, VOCAB), jnp.float32),
        scratch_shapes=[pltpu.VMEM((ROWS, PROJ), jnp.float32)],
        compiler_params=pltpu.CompilerParams(
            dimension_semantics=("arbitrary",),
        ),
    )(t2, W1, b1.reshape(1, PROJ), W2, b2.reshape(1, VOCAB))
    return out.reshape(B, S, VOCAB)
